# baseline (speedup 1.0000x reference)
import functools

import jax
import jax.numpy as jnp
from jax import lax
from jax.experimental import pallas as pl
from jax.experimental.pallas import tpu as pltpu
from jax.experimental.pallas import tpu_sc as plsc

B = 4096
L = 200
D = 64
NC = 2
NS = 16
NW = NC * NS
CH = 128
NCHUNK = (B * L) // CH
CPW = NCHUNK // NW


def _emb_grid(idx, table, pe):
    mesh = plsc.VectorSubcoreMesh(core_axis_name="c", subcore_axis_name="s")

    @functools.partial(
        pl.kernel,
        out_type=jax.ShapeDtypeStruct((B * L, D), jnp.float32),
        mesh=mesh,
        scratch_types=[
            pltpu.VMEM((CH,), jnp.int32),
            pltpu.VMEM((CH, D), jnp.float32),
            pltpu.VMEM((2 * L, D), jnp.float32),
            pltpu.SemaphoreType.DMA,
        ],
        compiler_params=pltpu.CompilerParams(use_tc_tiling_on_sc=False),
    )
    def k(idx_hbm, table_hbm, pe_hbm, out_hbm, idx_v, rows_v, pe2_v, sem):
        wid = lax.axis_index("s") * NC + lax.axis_index("c")

        pltpu.sync_copy(pe_hbm.at[pl.ds(0, L)], pe2_v.at[pl.ds(0, L)])
        pltpu.sync_copy(pe_hbm.at[pl.ds(0, L)], pe2_v.at[pl.ds(L, L)])

        def body(c, carry):
            chunk = wid * CPW + c
            pltpu.sync_copy(idx_hbm.at[chunk], idx_v)
            pltpu.async_copy(table_hbm.at[idx_v], rows_v, sem).wait()
            l0 = lax.rem(c * CH, L)
            for j in range(CH):
                for d in range(D // 16):
                    sl = pl.ds(d * 16, 16)
                    rows_v[j, sl] = rows_v[j, sl] + pe2_v[l0 + j, sl]
            pltpu.sync_copy(rows_v, out_hbm.at[pl.ds(chunk * CH, CH)])
            return carry

        lax.fori_loop(0, CPW, body, 0)

    return k(idx, table, pe)


def kernel(sequence, table, pe):
    idx = sequence.reshape(NCHUNK, CH)
    out = _emb_grid(idx, table, pe)
    return out.reshape(B, L, D)


# speedup vs baseline: 1.5934x; 1.5934x over previous
import functools

import jax
import jax.numpy as jnp
from jax import lax
from jax.experimental import pallas as pl
from jax.experimental.pallas import tpu as pltpu
from jax.experimental.pallas import tpu_sc as plsc

B = 4096
L = 200
D = 64
NC = 2
NS = 16
NW = NC * NS
CH = 128
NCHUNK = (B * L) // CH
CPW = NCHUNK // NW
NBUF = 4


def _emb_grid(idx, table, pe):
    mesh = plsc.VectorSubcoreMesh(core_axis_name="c", subcore_axis_name="s")

    @functools.partial(
        pl.kernel,
        out_type=jax.ShapeDtypeStruct((B * L, D), jnp.float32),
        mesh=mesh,
        scratch_types=(
            [pltpu.VMEM((CH,), jnp.int32) for _ in range(NBUF)]
            + [pltpu.VMEM((CH, D), jnp.float32) for _ in range(NBUF)]
            + [pltpu.VMEM((2 * L, D), jnp.float32)]
            + [pltpu.SemaphoreType.DMA for _ in range(3 * NBUF)]
        ),
        compiler_params=pltpu.CompilerParams(use_tc_tiling_on_sc=False),
    )
    def k(idx_hbm, table_hbm, pe_hbm, out_hbm, *sc):
        ibuf = sc[0:NBUF]
        rows = sc[NBUF:2 * NBUF]
        pe2_v = sc[2 * NBUF]
        gsem = sc[2 * NBUF + 1:2 * NBUF + 5]
        ssem = sc[2 * NBUF + 5:2 * NBUF + 9]
        isem = sc[2 * NBUF + 9:2 * NBUF + 13]

        wid = lax.axis_index("s") * NC + lax.axis_index("c")
        base = wid * CPW

        pltpu.sync_copy(pe_hbm.at[pl.ds(0, L)], pe2_v.at[pl.ds(0, L)])
        pltpu.sync_copy(pe_hbm.at[pl.ds(0, L)], pe2_v.at[pl.ds(L, L)])

        pltpu.sync_copy(idx_hbm.at[base], ibuf[0])
        pltpu.sync_copy(idx_hbm.at[base + 1], ibuf[1])
        pltpu.async_copy(idx_hbm.at[base + 2], ibuf[2], isem[2])
        pltpu.async_copy(table_hbm.at[ibuf[0]], rows[0], gsem[0])
        pltpu.async_copy(table_hbm.at[ibuf[1]], rows[1], gsem[1])

        def body(p, carry):
            for b in range(NBUF):
                c = p * NBUF + b
                kn = (b + 3) % NBUF
                kg = (b + 2) % NBUF
                pltpu.async_copy(idx_hbm.at[base + c + 3], ibuf[kn], isem[kn])

                def wait_scatter(kk=kg):
                    pltpu.make_async_copy(
                        rows[kk], out_hbm.at[pl.ds(0, CH)], ssem[kk]).wait()
                if b < 2:
                    pl.when(p > 0)(wait_scatter)
                else:
                    wait_scatter()

                pltpu.make_async_copy(idx_hbm.at[base], ibuf[kg], isem[kg]).wait()
                pltpu.async_copy(table_hbm.at[ibuf[kg]], rows[kg], gsem[kg])

                pltpu.make_async_copy(table_hbm.at[ibuf[b]], rows[b], gsem[b]).wait()
                l0 = lax.rem(c * CH, L)
                rv = rows[b]

                @plsc.parallel_loop(0, CH, step=1, unroll=8)
                def add_body(j):
                    for d in range(D // 16):
                        sl = pl.ds(d * 16, 16)
                        rv[j, sl] = rv[j, sl] + pe2_v[l0 + j, sl]

                pltpu.async_copy(
                    rv, out_hbm.at[pl.ds((base + c) * CH, CH)], ssem[b])
            return carry

        lax.fori_loop(0, CPW // NBUF, body, 0)

        pltpu.make_async_copy(rows[2], out_hbm.at[pl.ds(0, CH)], ssem[2]).wait()
        pltpu.make_async_copy(rows[3], out_hbm.at[pl.ds(0, CH)], ssem[3]).wait()
        pltpu.make_async_copy(table_hbm.at[ibuf[0]], rows[0], gsem[0]).wait()
        pltpu.make_async_copy(table_hbm.at[ibuf[1]], rows[1], gsem[1]).wait()
        pltpu.make_async_copy(idx_hbm.at[base], ibuf[2], isem[2]).wait()

    return k(idx, table, pe)


def kernel(sequence, table, pe):
    flat = sequence.reshape(-1)
    flat = jnp.concatenate([flat, jnp.zeros((NBUF * CH,), jnp.int32)])
    idx = flat.reshape(NCHUNK + NBUF, CH)
    out = _emb_grid(idx, table, pe)
    return out.reshape(B, L, D)


# speedup vs baseline: 5.5427x; 3.4785x over previous
import functools

import jax
import jax.numpy as jnp
from jax import lax
from jax.experimental import pallas as pl
from jax.experimental.pallas import tpu as pltpu
from jax.experimental.pallas import tpu_sc as plsc

B = 4096
L = 200
D = 64
VOCAB = 1000000
NC = 2
NS = 16
NW = NC * NS
CH = 128
BB = B // CH
NUNIT = L * BB
UPW = NUNIT // NW
NBUF = 4
DQ = D // 8

TBLK = VOCAB // 128
TPW = 245
TIPW = 248
TAIL = VOCAB - TBLK * 128


def _transpose_table(table_t, tail32):
    mesh = plsc.VectorSubcoreMesh(core_axis_name="c", subcore_axis_name="s")

    @functools.partial(
        pl.kernel,
        out_type=jax.ShapeDtypeStruct((VOCAB // 2, 128), jnp.float32),
        mesh=mesh,
        scratch_types=(
            [pltpu.VMEM((D, 128), jnp.float32) for _ in range(4)]
            + [pltpu.VMEM((64, 128), jnp.float32) for _ in range(2)]
            + [pltpu.VMEM((TAIL // 2, 128), jnp.float32)]
            + [pltpu.SemaphoreType.DMA for _ in range(6)]
        ),
        compiler_params=pltpu.CompilerParams(
            use_tc_tiling_on_sc=True, needs_layout_passes=False),
    )
    def ka(tt_hbm, tail_hbm, pk_hbm, *sc):
        inb = sc[0:4]
        oub = sc[4:6]
        tl_v = sc[6]
        isem = sc[7:11]
        osem = sc[11:13]

        wid = lax.axis_index("s") * NC + lax.axis_index("c")
        t0 = wid * TPW
        iota16 = lax.iota(jnp.int32, 16)
        cidx = [iota16 + 16 * ci for ci in range(8)]
        ridx = [(iota16 + 16 * ci) // 2 for ci in range(8)]
        parb = [(iota16 & 1) * 64 for _ in range(8)]

        def start_in(i, s):
            v0 = pl.multiple_of(
                jnp.minimum((t0 + i) * 128, (TBLK - 1) * 128), 128)
            pltpu.async_copy(tt_hbm.at[:, pl.ds(v0, 128)], inb[s], isem[s])

        start_in(0, 0)
        start_in(1, 1)
        start_in(2, 2)

        def body(p, carry):
            for s in range(4):
                i = 4 * p + s
                start_in(i + 3, (s + 3) % 4)
                pltpu.make_async_copy(
                    tt_hbm.at[:, pl.ds(0, 128)], inb[s], isem[s]).wait()

                def wait_out(ss=s % 2):
                    pltpu.make_async_copy(
                        oub[ss], pk_hbm.at[pl.ds(0, 64), :], osem[ss]).wait()
                if s < 2:
                    pl.when(p > 0)(wait_out)
                else:
                    wait_out()

                v0 = jnp.minimum((t0 + i) * 128, (TBLK - 1) * 128)
                ib = inb[s]
                ob = oub[s % 2]

                @plsc.parallel_loop(0, D, step=1, unroll=2)
                def tr(d0):
                    dmask = (d0 + iota16) & 63
                    for ci in range(8):
                        v = plsc.load_gather(ib, [dmask, cidx[ci]])
                        plsc.store_scatter(ob, [ridx[ci], parb[ci] + dmask], v)

                pltpu.async_copy(
                    ob, pk_hbm.at[pl.ds(pl.multiple_of(v0 // 2, 64), 64), :],
                    osem[s % 2])
            return carry

        lax.fori_loop(0, TIPW // 4, body, 0)

        for s in range(2):
            pltpu.make_async_copy(
                oub[s], pk_hbm.at[pl.ds(0, 64), :], osem[s]).wait()
        for s in range(3):
            pltpu.make_async_copy(
                tt_hbm.at[:, pl.ds(0, 128)], inb[s], isem[s]).wait()

        @pl.when(wid == 0)
        def _():
            pltpu.sync_copy(tail_hbm, tl_v)
            pltpu.sync_copy(tl_v, pk_hbm.at[pl.ds(VOCAB // 2 - TAIL // 2,
                                                  TAIL // 2), :])

    return ka(table_t, tail32)


def _emb_grid(idx, table_lin, pe):
    mesh = plsc.VectorSubcoreMesh(core_axis_name="c", subcore_axis_name="s")

    @functools.partial(
        pl.kernel,
        out_type=jax.ShapeDtypeStruct((L, DQ, BB, 8, CH), jnp.float32),
        mesh=mesh,
        scratch_types=(
            [pltpu.VMEM((CH,), jnp.int32) for _ in range(NBUF)]
            + [pltpu.VMEM((CH, D), jnp.float32) for _ in range(NBUF)]
            + [pltpu.VMEM((D, CH + 1), jnp.float32) for _ in range(NBUF)]
            + [pltpu.VMEM((L, D), jnp.float32)]
            + [pltpu.SemaphoreType.DMA for _ in range(3 * NBUF)]
        ),
        compiler_params=pltpu.CompilerParams(
            use_tc_tiling_on_sc=False, needs_layout_passes=False),
    )
    def k(idx_hbm, table_hbm, pe_hbm, out_hbm, *sc):
        ibuf = sc[0:NBUF]
        rows = sc[NBUF:2 * NBUF]
        outb = sc[2 * NBUF:3 * NBUF]
        pe_v = sc[3 * NBUF]
        gsem = sc[3 * NBUF + 1:4 * NBUF + 1]
        isem = sc[4 * NBUF + 1:5 * NBUF + 1]
        ssem = sc[5 * NBUF + 1:6 * NBUF + 1]

        wid = lax.axis_index("s") * NC + lax.axis_index("c")
        base = wid * UPW
        iota16 = lax.iota(jnp.int32, 16)

        pltpu.sync_copy(pe_hbm.at[pl.ds(0, L)], pe_v)

        pltpu.sync_copy(idx_hbm.at[base], ibuf[0])
        pltpu.sync_copy(idx_hbm.at[base + 1], ibuf[1])
        pltpu.async_copy(idx_hbm.at[base + 2], ibuf[2], isem[2])
        pltpu.async_copy(table_hbm.at[ibuf[0]], rows[0], gsem[0])
        pltpu.async_copy(table_hbm.at[ibuf[1]], rows[1], gsem[1])

        def body(p, carry):
            for b in range(NBUF):
                c = p * NBUF + b
                u = base + c
                kn = (b + 3) % NBUF
                kg = (b + 2) % NBUF
                ob = b
                pltpu.async_copy(idx_hbm.at[u + 3], ibuf[kn], isem[kn])
                pltpu.make_async_copy(idx_hbm.at[base], ibuf[kg], isem[kg]).wait()
                pltpu.async_copy(table_hbm.at[ibuf[kg]], rows[kg], gsem[kg])
                pltpu.make_async_copy(table_hbm.at[ibuf[b]], rows[b], gsem[b]).wait()

                def wait_scatter(kk=ob):
                    for _ in range(DQ):
                        pltpu.make_async_copy(
                            outb[kk].at[pl.ds(0, 8), pl.ds(0, CH)],
                            out_hbm.at[0, 0, 0], ssem[kk]).wait()
                pl.when(p > 0)(wait_scatter)

                l = u // BB
                bq = lax.rem(u, BB)
                rv = rows[b]
                ov = outb[ob]
                pes = tuple(pe_v[l, pl.ds(q * 16, 16)] for q in range(4))

                @plsc.parallel_loop(0, CH, step=1, unroll=4, carry=pes)
                def add_t(j, pes):
                    colj = jnp.full((16,), j, jnp.int32)
                    for q in range(4):
                        v = rv[j, pl.ds(q * 16, 16)] + pes[q]
                        plsc.store_scatter(ov, [iota16 + q * 16, colj], v)
                    return pes

                for dq in range(DQ):
                    pltpu.async_copy(
                        ov.at[pl.ds(dq * 8, 8), pl.ds(0, CH)],
                        out_hbm.at[l, dq, bq], ssem[ob])
            return carry

        lax.fori_loop(0, UPW // NBUF, body, 0)

        for kk in range(NBUF):
            for _ in range(DQ):
                pltpu.make_async_copy(
                    outb[kk].at[pl.ds(0, 8), pl.ds(0, CH)],
                    out_hbm.at[0, 0, 0], ssem[kk]).wait()
        pltpu.make_async_copy(table_hbm.at[ibuf[0]], rows[0], gsem[0]).wait()
        pltpu.make_async_copy(table_hbm.at[ibuf[1]], rows[1], gsem[1]).wait()
        pltpu.make_async_copy(idx_hbm.at[base], ibuf[2], isem[2]).wait()

    return k(idx, table_lin, pe)


def kernel(sequence, table, pe):
    seq_t = jnp.swapaxes(sequence, 0, 1).reshape(-1)
    seq_t = jnp.concatenate([seq_t, jnp.zeros((NBUF * CH,), jnp.int32)])
    idx = seq_t.reshape(NUNIT + NBUF, CH)
    table_t = jnp.swapaxes(table, 0, 1)
    tail32 = table[TBLK * 128:].reshape(TAIL // 2, 128)
    pk = _transpose_table(table_t, tail32)
    out5 = _emb_grid(idx, pk.reshape(VOCAB, D), pe)
    return out5.transpose(2, 4, 0, 1, 3).reshape(B, L, D)


# speedup vs baseline: 5.5567x; 1.0025x over previous
import functools

import jax
import jax.numpy as jnp
from jax import lax
from jax.experimental import pallas as pl
from jax.experimental.pallas import tpu as pltpu
from jax.experimental.pallas import tpu_sc as plsc

B = 4096
L = 200
D = 64
VOCAB = 1000000
NC = 2
NS = 16
NW = NC * NS
CH = 128
BB = B // CH
NUNIT = L * BB
UPW = NUNIT // NW
NBUF = 4
DQ = D // 8

TBLK = VOCAB // 128
TPW = 245
TIPW = 248
TAIL = VOCAB - TBLK * 128


def _transpose_table(table_t, tail32):
    mesh = plsc.VectorSubcoreMesh(core_axis_name="c", subcore_axis_name="s")

    @functools.partial(
        pl.kernel,
        out_type=jax.ShapeDtypeStruct((VOCAB // 2, 128), jnp.float32),
        mesh=mesh,
        scratch_types=(
            [pltpu.VMEM((D, 128), jnp.float32) for _ in range(4)]
            + [pltpu.VMEM((64, 128), jnp.float32) for _ in range(2)]
            + [pltpu.VMEM((TAIL // 2, 128), jnp.float32)]
            + [pltpu.SemaphoreType.DMA for _ in range(6)]
        ),
        compiler_params=pltpu.CompilerParams(
            use_tc_tiling_on_sc=True, needs_layout_passes=False),
    )
    def ka(tt_hbm, tail_hbm, pk_hbm, *sc):
        inb = sc[0:4]
        oub = sc[4:6]
        tl_v = sc[6]
        isem = sc[7:11]
        osem = sc[11:13]

        wid = lax.axis_index("s") * NC + lax.axis_index("c")
        t0 = wid * TPW
        iota16 = lax.iota(jnp.int32, 16)
        cidx = [iota16 + 16 * ci for ci in range(8)]
        ridx = [(iota16 + 16 * ci) // 2 for ci in range(8)]
        parb = [(iota16 & 1) * 64 for _ in range(8)]

        def start_in(i, s):
            v0 = pl.multiple_of(
                jnp.minimum((t0 + i) * 128, (TBLK - 1) * 128), 128)
            pltpu.async_copy(tt_hbm.at[:, pl.ds(v0, 128)], inb[s], isem[s])

        start_in(0, 0)
        start_in(1, 1)
        start_in(2, 2)

        def body(p, carry):
            for s in range(4):
                i = 4 * p + s
                start_in(i + 3, (s + 3) % 4)
                pltpu.make_async_copy(
                    tt_hbm.at[:, pl.ds(0, 128)], inb[s], isem[s]).wait()

                def wait_out(ss=s % 2):
                    pltpu.make_async_copy(
                        oub[ss], pk_hbm.at[pl.ds(0, 64), :], osem[ss]).wait()
                if s < 2:
                    pl.when(p > 0)(wait_out)
                else:
                    wait_out()

                v0 = jnp.minimum((t0 + i) * 128, (TBLK - 1) * 128)
                ib = inb[s]
                ob = oub[s % 2]

                @plsc.parallel_loop(0, D, step=1, unroll=4)
                def tr(d0):
                    dmask = (d0 + iota16) & 63
                    for ci in range(8):
                        v = plsc.load_gather(ib, [dmask, cidx[ci]])
                        plsc.store_scatter(ob, [ridx[ci], parb[ci] + dmask], v)

                pltpu.async_copy(
                    ob, pk_hbm.at[pl.ds(pl.multiple_of(v0 // 2, 64), 64), :],
                    osem[s % 2])
            return carry

        lax.fori_loop(0, TIPW // 4, body, 0)

        for s in range(2):
            pltpu.make_async_copy(
                oub[s], pk_hbm.at[pl.ds(0, 64), :], osem[s]).wait()
        for s in range(3):
            pltpu.make_async_copy(
                tt_hbm.at[:, pl.ds(0, 128)], inb[s], isem[s]).wait()

        @pl.when(wid == 0)
        def _():
            pltpu.sync_copy(tail_hbm, tl_v)
            pltpu.sync_copy(tl_v, pk_hbm.at[pl.ds(VOCAB // 2 - TAIL // 2,
                                                  TAIL // 2), :])

    return ka(table_t, tail32)


def _emb_grid(idx, table_lin, pe):
    mesh = plsc.VectorSubcoreMesh(core_axis_name="c", subcore_axis_name="s")

    @functools.partial(
        pl.kernel,
        out_type=jax.ShapeDtypeStruct((L, DQ, BB, 8, CH), jnp.float32),
        mesh=mesh,
        scratch_types=(
            [pltpu.VMEM((CH,), jnp.int32) for _ in range(NBUF)]
            + [pltpu.VMEM((CH, D), jnp.float32) for _ in range(NBUF)]
            + [pltpu.VMEM((D, CH + 1), jnp.float32) for _ in range(NBUF)]
            + [pltpu.VMEM((L, D), jnp.float32)]
            + [pltpu.SemaphoreType.DMA for _ in range(3 * NBUF)]
        ),
        compiler_params=pltpu.CompilerParams(
            use_tc_tiling_on_sc=False, needs_layout_passes=False),
    )
    def k(idx_hbm, table_hbm, pe_hbm, out_hbm, *sc):
        ibuf = sc[0:NBUF]
        rows = sc[NBUF:2 * NBUF]
        outb = sc[2 * NBUF:3 * NBUF]
        pe_v = sc[3 * NBUF]
        gsem = sc[3 * NBUF + 1:4 * NBUF + 1]
        isem = sc[4 * NBUF + 1:5 * NBUF + 1]
        ssem = sc[5 * NBUF + 1:6 * NBUF + 1]

        wid = lax.axis_index("s") * NC + lax.axis_index("c")
        base = wid * UPW
        iota16 = lax.iota(jnp.int32, 16)

        pltpu.sync_copy(pe_hbm.at[pl.ds(0, L)], pe_v)

        pltpu.sync_copy(idx_hbm.at[base], ibuf[0])
        pltpu.sync_copy(idx_hbm.at[base + 1], ibuf[1])
        pltpu.async_copy(idx_hbm.at[base + 2], ibuf[2], isem[2])
        pltpu.async_copy(table_hbm.at[ibuf[0]], rows[0], gsem[0])
        pltpu.async_copy(table_hbm.at[ibuf[1]], rows[1], gsem[1])

        def body(p, carry):
            for b in range(NBUF):
                c = p * NBUF + b
                u = base + c
                kn = (b + 3) % NBUF
                kg = (b + 2) % NBUF
                ob = b
                pltpu.async_copy(idx_hbm.at[u + 3], ibuf[kn], isem[kn])
                pltpu.make_async_copy(idx_hbm.at[base], ibuf[kg], isem[kg]).wait()
                pltpu.async_copy(table_hbm.at[ibuf[kg]], rows[kg], gsem[kg])
                pltpu.make_async_copy(table_hbm.at[ibuf[b]], rows[b], gsem[b]).wait()

                def wait_scatter(kk=ob):
                    for _ in range(DQ):
                        pltpu.make_async_copy(
                            outb[kk].at[pl.ds(0, 8), pl.ds(0, CH)],
                            out_hbm.at[0, 0, 0], ssem[kk]).wait()
                pl.when(p > 0)(wait_scatter)

                l = u // BB
                bq = lax.rem(u, BB)
                rv = rows[b]
                ov = outb[ob]
                pes = tuple(pe_v[l, pl.ds(q * 16, 16)] for q in range(4))

                @plsc.parallel_loop(0, CH, step=1, unroll=4, carry=pes)
                def add_t(j, pes):
                    colj = jnp.full((16,), j, jnp.int32)
                    for q in range(4):
                        v = rv[j, pl.ds(q * 16, 16)] + pes[q]
                        plsc.store_scatter(ov, [iota16 + q * 16, colj], v)
                    return pes

                for dq in range(DQ):
                    pltpu.async_copy(
                        ov.at[pl.ds(dq * 8, 8), pl.ds(0, CH)],
                        out_hbm.at[l, dq, bq], ssem[ob])
            return carry

        lax.fori_loop(0, UPW // NBUF, body, 0)

        for kk in range(NBUF):
            for _ in range(DQ):
                pltpu.make_async_copy(
                    outb[kk].at[pl.ds(0, 8), pl.ds(0, CH)],
                    out_hbm.at[0, 0, 0], ssem[kk]).wait()
        pltpu.make_async_copy(table_hbm.at[ibuf[0]], rows[0], gsem[0]).wait()
        pltpu.make_async_copy(table_hbm.at[ibuf[1]], rows[1], gsem[1]).wait()
        pltpu.make_async_copy(idx_hbm.at[base], ibuf[2], isem[2]).wait()

    return k(idx, table_lin, pe)


def kernel(sequence, table, pe):
    seq_t = jnp.swapaxes(sequence, 0, 1).reshape(-1)
    seq_t = jnp.concatenate([seq_t, jnp.zeros((NBUF * CH,), jnp.int32)])
    idx = seq_t.reshape(NUNIT + NBUF, CH)
    table_t = jnp.swapaxes(table, 0, 1)
    tail32 = table[TBLK * 128:].reshape(TAIL // 2, 128)
    pk = _transpose_table(table_t, tail32)
    out5 = _emb_grid(idx, pk.reshape(VOCAB, D), pe)
    return out5.transpose(2, 4, 0, 1, 3).reshape(B, L, D)
